# baseline (speedup 1.0000x reference)
import functools

import jax
import jax.numpy as jnp
from jax import lax
from jax.experimental import pallas as pl
from jax.experimental.pallas import tpu as pltpu
from jax.experimental.pallas import tpu_sc as plsc

_B = 16384
_D = 64
_NNEG = 5
_V = 1000000

_info = plsc.get_sparse_core_info()
_NC = _info.num_cores
_NS = _info.num_subcores
_NW = _NC * _NS

_PC_PER_W = _B // _NW
_NEG_PER_W = _B * _NNEG // _NW
_TOT_PER_W = 2 * _PC_PER_W + _NEG_PER_W
_G = 64
_L = 16


def _sc_body(centerP, contextP, pc_idx_hbm, pctx_idx_hbm, neg_idx_hbm,
             pc_out, pctx_out, neg_out3,
             idx_v, jbuf, rows_a, rows_b, stage2, stage3, sem_a, sem_b):
    w = lax.axis_index("s") * _NC + lax.axis_index("c")
    base = w * _PC_PER_W
    nbase = w * _NEG_PER_W

    pltpu.sync_copy(pc_idx_hbm.at[pl.ds(base, _PC_PER_W)],
                    idx_v.at[pl.ds(0, _PC_PER_W)])
    pltpu.sync_copy(pctx_idx_hbm.at[pl.ds(base, _PC_PER_W)],
                    idx_v.at[pl.ds(_PC_PER_W, _PC_PER_W)])
    pltpu.sync_copy(neg_idx_hbm.at[pl.ds(nbase, _NEG_PER_W)],
                    idx_v.at[pl.ds(2 * _PC_PER_W, _NEG_PER_W)])

    def mkj(t, carry):
        r = t // 4
        o = (t % 4) * _L
        jbuf[r, pl.ds(o, _L)] = idx_v[pl.ds(t * _L, _L)] >> 1
        return carry
    lax.fori_loop(0, _TOT_PER_W // _L, mkj, 0)

    lanes = lax.iota(jnp.int32, _L)
    bufs = (rows_a, rows_b)
    sems = (sem_a, sem_b)

    def gather(tableP, jr, s):
        return pltpu.async_copy(tableP.at[jbuf.at[jr + s]],
                                bufs[s % 2], sems[s % 2])

    def extract(flat_off, s, put):
        rv = bufs[s % 2]

        def group(g, carry):
            iv = idx_v[pl.ds(flat_off + s * _G + g * _L, _L)]
            par = (iv & 1) * _D
            rows = g * _L + lanes
            for c in range(_D):
                vals = plsc.load_gather(rv, [rows, par + c])
                put(s, rows, c, vals)
            return carry

        lax.fori_loop(0, _G // _L, group, 0)

    def run_chunk(tableP, jr, flat_off, n_sub, put):
        h = gather(tableP, jr, 0)
        hs = [h]
        for s in range(1, n_sub + 1):
            if s < n_sub:
                hs.append(gather(tableP, jr, s))
            hs[s - 1].wait()
            extract(flat_off, s - 1, put)

    def put2(s, rows, c, vals):
        plsc.store_scatter(stage2, [s * _G + rows,
                                    jnp.full((_L,), c, jnp.int32)], vals)

    def flat_section(tableP, jr0, flat0, out_hbm, out_base):
        def chunk(c, carry):
            run_chunk(tableP, jr0 + 2 * c, flat0 + 128 * c, 2, put2)
            pltpu.sync_copy(stage2, out_hbm.at[pl.ds(out_base + 128 * c, 128)])
            return carry
        lax.fori_loop(0, _PC_PER_W // 128, chunk, 0)

    flat_section(centerP, 0, 0, pc_out, base)
    flat_section(contextP, _PC_PER_W // _G, _PC_PER_W, pctx_out, base)

    stage3d = stage3.reshape(_G, _NNEG, _D)

    def put3(s, rows, c, vals):
        plsc.store_scatter(stage3, [s * _G + rows,
                                    jnp.full((_L,), c, jnp.int32)], vals)

    def neg_chunk(c, carry):
        run_chunk(contextP, 2 * _PC_PER_W // _G + _NNEG * c,
                  2 * _PC_PER_W + _NNEG * _G * c, _NNEG, put3)
        pltpu.sync_copy(stage3d, neg_out3.at[pl.ds(w * _PC_PER_W + _G * c, _G)])
        return carry
    lax.fori_loop(0, _PC_PER_W // _G, neg_chunk, 0)


@jax.jit
def _fasttext_gather(center_W, context_W, pc_idx, pctx_idx, neg_idx):
    centerP = center_W.reshape(_V // 2, 2 * _D)
    contextP = context_W.reshape(_V // 2, 2 * _D)
    mesh = plsc.VectorSubcoreMesh(core_axis_name="c", subcore_axis_name="s")
    return pl.kernel(
        _sc_body,
        mesh=mesh,
        compiler_params=pltpu.CompilerParams(needs_layout_passes=False),
        out_type=(
            jax.ShapeDtypeStruct((_B, _D), jnp.float32),
            jax.ShapeDtypeStruct((_B, _D), jnp.float32),
            jax.ShapeDtypeStruct((_B, _NNEG, _D), jnp.float32),
        ),
        scratch_types=[
            pltpu.VMEM((_TOT_PER_W,), jnp.int32),
            pltpu.VMEM((_TOT_PER_W // _G, _G), jnp.int32),
            pltpu.VMEM((_G, 2 * _D), jnp.float32),
            pltpu.VMEM((_G, 2 * _D), jnp.float32),
            pltpu.VMEM((128, _D), jnp.float32),
            pltpu.VMEM((_G * _NNEG, _D), jnp.float32),
            pltpu.SemaphoreType.DMA,
            pltpu.SemaphoreType.DMA,
        ],
    )(centerP, contextP, pc_idx, pctx_idx, neg_idx)


def kernel(center_W, context_W, pos_center, pos_context, neg_context):
    pc_idx = pos_center.astype(jnp.int32)
    pctx_idx = pos_context.astype(jnp.int32)
    neg_idx = neg_context.reshape(-1).astype(jnp.int32)
    return _fasttext_gather(center_W, context_W, pc_idx, pctx_idx, neg_idx)


# speedup vs baseline: 2.3951x; 2.3951x over previous
import functools

import jax
import jax.numpy as jnp
from jax import lax
from jax.experimental import pallas as pl
from jax.experimental.pallas import tpu as pltpu
from jax.experimental.pallas import tpu_sc as plsc

_B = 16384
_D = 64
_NNEG = 5
_V = 1000000

_info = plsc.get_sparse_core_info()
_NC = _info.num_cores
_NS = _info.num_subcores
_NW = _NC * _NS

_PC_PER_W = _B // _NW
_NEG_PER_W = _B * _NNEG // _NW
_CHUNK = 256

_MESH = plsc.VectorSubcoreMesh(core_axis_name="c", subcore_axis_name="s")


def _wid():
    return lax.axis_index("s") * _NC + lax.axis_index("c")


def _row_dmas(table3, idx_v, coff, n, dst_for, sem):
    copies = []
    for k in range(n):
        if k % 16 == 0:
            iv = idx_v[pl.ds(coff + k, 16)]
        i = iv[k % 16]
        copies.append(pltpu.async_copy(
            table3.at[i >> 3, i & 7], dst_for(k), sem))
    for h in copies:
        h.wait()


def _flat_section(table3, idx_v, idx_off, out_hbm, out_base, n_chunks,
                  rows_v, sem):
    def chunk(c, carry):
        _row_dmas(table3, idx_v, idx_off + c * _CHUNK, _CHUNK,
                  lambda k: rows_v.at[k], sem)
        pltpu.sync_copy(rows_v, out_hbm.at[pl.ds(out_base + c * _CHUNK, _CHUNK)])
        return carry

    lax.fori_loop(0, n_chunks, chunk, 0)


def _ctx_body(context3, pctx_idx_hbm, neg_idx_hbm, pctx_out, neg_out3,
              idx_v, rows_v, rows_v3, sem):
    w = _wid()
    base = w * _PC_PER_W
    nbase = w * _NEG_PER_W

    pltpu.sync_copy(pctx_idx_hbm.at[pl.ds(base, _PC_PER_W)],
                    idx_v.at[pl.ds(0, _PC_PER_W)])
    pltpu.sync_copy(neg_idx_hbm.at[pl.ds(nbase, _NEG_PER_W)],
                    idx_v.at[pl.ds(_PC_PER_W, _NEG_PER_W)])

    _flat_section(context3, idx_v, 0, pctx_out, base, _PC_PER_W // _CHUNK,
                  rows_v, sem)

    def neg_chunk(c, carry):
        _row_dmas(context3, idx_v, _PC_PER_W + c * (_NNEG * 64), _NNEG * 64,
                  lambda k: rows_v3.at[k // _NNEG, k % _NNEG], sem)
        pltpu.sync_copy(rows_v3, neg_out3.at[pl.ds(base + c * 64, 64)])
        return carry

    lax.fori_loop(0, _PC_PER_W // 64, neg_chunk, 0)


def _pc_body(center_hbm, pc_idx_hbm, pc_out, idx_v, rows_v, sem):
    center3 = center_hbm.reshape(_V // 8, 8, _D)
    base = _wid() * _PC_PER_W
    pltpu.sync_copy(pc_idx_hbm.at[pl.ds(base, _PC_PER_W)], idx_v)
    _flat_section(center3, idx_v, 0, pc_out, base, _PC_PER_W // _CHUNK,
                  rows_v, sem)


@jax.jit
def _fasttext_gather(center_W, context_W, pc_idx, pctx_idx, neg_idx):
    context3 = context_W.reshape(_V // 8, 8, _D)
    pctx, nctx = pl.kernel(
        _ctx_body,
        mesh=_MESH,
        out_type=(
            jax.ShapeDtypeStruct((_B, _D), jnp.float32),
            jax.ShapeDtypeStruct((_B, _NNEG, _D), jnp.float32),
        ),
        scratch_types=[
            pltpu.VMEM((_PC_PER_W + _NEG_PER_W,), jnp.int32),
            pltpu.VMEM((_CHUNK, _D), jnp.float32),
            pltpu.VMEM((64, _NNEG, _D), jnp.float32),
            pltpu.SemaphoreType.DMA,
        ],
    )(context3, pctx_idx, neg_idx)
    pc = pl.kernel(
        _pc_body,
        mesh=_MESH,
        out_type=jax.ShapeDtypeStruct((_B, _D), jnp.float32),
        scratch_types=[
            pltpu.VMEM((_PC_PER_W,), jnp.int32),
            pltpu.VMEM((_CHUNK, _D), jnp.float32),
            pltpu.SemaphoreType.DMA,
        ],
    )(center_W, pc_idx)
    return pc, pctx, nctx


def kernel(center_W, context_W, pos_center, pos_context, neg_context):
    pc_idx = pos_center.astype(jnp.int32)
    pctx_idx = pos_context.astype(jnp.int32)
    neg_idx = neg_context.reshape(-1).astype(jnp.int32)
    return _fasttext_gather(center_W, context_W, pc_idx, pctx_idx, neg_idx)


# speedup vs baseline: 2.5953x; 1.0836x over previous
import functools

import jax
import jax.numpy as jnp
from jax import lax
from jax.experimental import pallas as pl
from jax.experimental.pallas import tpu as pltpu
from jax.experimental.pallas import tpu_sc as plsc

_B = 16384
_D = 64
_NNEG = 5
_V = 1000000

_info = plsc.get_sparse_core_info()
_NC = _info.num_cores
_NS = _info.num_subcores
_NW = _NC * _NS

_PC_PER_W = _B // _NW
_NEG_PER_W = _B * _NNEG // _NW
_CHUNK = 256


def _row_dmas(table3, idx_v, coff, n, dst_for, sem):
    for k in range(n):
        if k % 16 == 0:
            iv = idx_v[pl.ds(coff + k, 16)]
        i = iv[k % 16]
        pltpu.async_copy(table3.at[i >> 3, i & 7], dst_for(k), sem)


def _drain(dummy_src, whole_dst, sem):
    pltpu.make_async_copy(dummy_src, whole_dst, sem).wait()


def _sc_body(center3, context3, pc_idx_hbm, pctx_idx_hbm, neg_idx_hbm,
             pc_out, pctx_out, neg_out3,
             idx_v, rows_v, rows_v3, sem):
    w = lax.axis_index("s") * _NC + lax.axis_index("c")
    base = w * _PC_PER_W
    nbase = w * _NEG_PER_W

    pltpu.sync_copy(pc_idx_hbm.at[pl.ds(base, _PC_PER_W)],
                    idx_v.at[pl.ds(0, _PC_PER_W)])
    pltpu.sync_copy(pctx_idx_hbm.at[pl.ds(base, _PC_PER_W)],
                    idx_v.at[pl.ds(_PC_PER_W, _PC_PER_W)])
    pltpu.sync_copy(neg_idx_hbm.at[pl.ds(nbase, _NEG_PER_W)],
                    idx_v.at[pl.ds(2 * _PC_PER_W, _NEG_PER_W)])

    def flat_section(table3, idx_off, out_hbm, out_base):
        def chunk(c, carry):
            _row_dmas(table3, idx_v, idx_off + c * _CHUNK, _CHUNK,
                      lambda k: rows_v.at[k, pl.ds(0, _D)], sem)
            _drain(out_hbm.at[pl.ds(0, _CHUNK)], rows_v.at[:, pl.ds(0, _D)],
                   sem)
            pltpu.sync_copy(rows_v.at[:, pl.ds(0, _D)],
                            out_hbm.at[pl.ds(out_base + c * _CHUNK, _CHUNK)])
            return carry
        lax.fori_loop(0, _PC_PER_W // _CHUNK, chunk, 0)

    flat_section(center3, 0, pc_out, base)
    flat_section(context3, _PC_PER_W, pctx_out, base)

    def neg_chunk(c, carry):
        _row_dmas(context3, idx_v, 2 * _PC_PER_W + c * (_NNEG * 64),
                  _NNEG * 64,
                  lambda k: rows_v3.at[k // _NNEG, k % _NNEG], sem)
        _drain(neg_out3.at[pl.ds(0, 64)], rows_v3, sem)
        pltpu.sync_copy(rows_v3, neg_out3.at[pl.ds(base + c * 64, 64)])
        return carry
    lax.fori_loop(0, _PC_PER_W // 64, neg_chunk, 0)


@jax.jit
def _fasttext_gather(center_W, context_W, pc_idx, pctx_idx, neg_idx):
    center3 = center_W.reshape(_V // 8, 8, _D)
    context3 = context_W.reshape(_V // 8, 8, _D)
    mesh = plsc.VectorSubcoreMesh(core_axis_name="c", subcore_axis_name="s")
    return pl.kernel(
        _sc_body,
        mesh=mesh,
        out_type=(
            jax.ShapeDtypeStruct((_B, _D), jnp.float32),
            jax.ShapeDtypeStruct((_B, _D), jnp.float32),
            jax.ShapeDtypeStruct((_B, _NNEG, _D), jnp.float32),
        ),
        scratch_types=[
            pltpu.VMEM((2 * _PC_PER_W + _NEG_PER_W,), jnp.int32),
            pltpu.VMEM((_CHUNK, _D), jnp.float32),
            pltpu.VMEM((64, _NNEG, _D), jnp.float32),
            pltpu.SemaphoreType.DMA,
        ],
    )(center3, context3, pc_idx, pctx_idx, neg_idx)


def kernel(center_W, context_W, pos_center, pos_context, neg_context):
    pc_idx = pos_center.astype(jnp.int32)
    pctx_idx = pos_context.astype(jnp.int32)
    neg_idx = neg_context.reshape(-1).astype(jnp.int32)
    return _fasttext_gather(center_W, context_W, pc_idx, pctx_idx, neg_idx)
